# baseline (speedup 1.0000x reference)
import jax
import jax.numpy as jnp
from jax import lax
from jax.experimental import pallas as pl
from jax.experimental.pallas import tpu as pltpu
from jax.experimental.pallas import tpu_sc as plsc

_NUM_FRAMES = 10000
_NUM_BBOXES = 512
_N = 1048576
_PLANE = _NUM_FRAMES * _NUM_BBOXES

_NC = 2
_NS = 16
_NW = _NC * _NS
_L = 16

_BPT = (_N // 128) // _NW
_NBLK = _N // 128

_CCH = 3200
_PPT = _PLANE // _NW
_CCHUNKS = _PPT // _CCH

_COS_C = [-1.0 / 87178291200.0, 1.0 / 479001600.0, -1.0 / 3628800.0,
          1.0 / 40320.0, -1.0 / 720.0, 1.0 / 24.0, -1.0 / 2.0, 1.0]
_SINC_C = [-1.0 / 1307674368000.0, 1.0 / 6227020800.0, -1.0 / 39916800.0,
           1.0 / 362880.0, -1.0 / 5040.0, 1.0 / 120.0, -1.0 / 6.0, 1.0]
_OMC_C = [-1.0 / 20922789888000.0, 1.0 / 87178291200.0, -1.0 / 479001600.0,
          1.0 / 3628800.0, -1.0 / 40320.0, 1.0 / 720.0, -1.0 / 24.0, 0.5]


def _horner(coeffs, u):
    acc = jnp.full((_L,), coeffs[0], jnp.float32)
    for c in coeffs[1:]:
        acc = acc * u + jnp.float32(c)
    return acc


def _convert_body(soa_hbm, aos_hbm, plane_v, aos_v, sem_i0, sem_i1,
                  sem_o0, sem_o1):
    wid = lax.axis_index("s") * _NC + lax.axis_index("c")
    iota = lax.broadcasted_iota(jnp.int32, (_L,), 0)
    sem_i = (sem_i0, sem_i1)
    sem_o = (sem_o0, sem_o1)
    p_base = wid * _PPT

    def in_descs(t, par):
        p0 = pl.multiple_of(p_base + t * _CCH, 8)
        return [pltpu.make_async_copy(
                    soa_hbm.at[pl.ds(c * _PLANE + p0, _CCH)],
                    plane_v.at[par * 6 + c], sem_i[par])
                for c in range(6)]

    def out_descs(t, par):
        p0 = pl.multiple_of(p_base + t * _CCH, 8)
        return [pltpu.make_async_copy(
                    aos_v.at[par], aos_hbm.at[pl.ds(p0 * 8, _CCH * 8)],
                    sem_o[par])]

    def fire(descs):
        for d in descs:
            d.start()

    def drain(descs):
        for d in descs:
            d.wait()

    fire(in_descs(0, 0))

    def step(s, carry):
        for k in range(2):
            t = s * 2 + k
            par = k
            nxt = 1 - k

            @pl.when(t + 1 < _CCHUNKS)
            def _():
                fire(in_descs(t + 1, nxt))
            drain(in_descs(t, par))

            @pl.when(t >= 2)
            def _():
                drain(out_descs(t - 2, par))

            def group(g, carry2):
                base = lax.shift_left(g * _L + iota, jnp.int32(3))
                for c in range(6):
                    v = plane_v[par * 6 + c, pl.ds(g * _L, _L)]
                    plsc.store_scatter(
                        aos_v, [jnp.full((_L,), par, jnp.int32),
                                base + jnp.int32(c)], v)
                return carry2

            lax.fori_loop(0, _CCH // _L, group, 0)
            fire(out_descs(t, par))
        return carry

    lax.fori_loop(0, _CCHUNKS // 2, step, 0)
    drain(out_descs(_CCHUNKS - 2, 0))
    drain(out_descs(_CCHUNKS - 1, 1))


def _gather_body(idx_hbm, aos_hbm, out_hbm, idx_v, rows_v, out_v,
                 sem_g0, sem_g1, sem_o0, sem_o1):
    wid = lax.axis_index("s") * _NC + lax.axis_index("c")
    iota = lax.broadcasted_iota(jnp.int32, (_L,), 0)
    sem_g = (sem_g0, sem_g1)
    sem_o = (sem_o0, sem_o1)
    bt0 = wid * _BPT

    pltpu.sync_copy(
        idx_hbm.at[pl.ds(pl.multiple_of(wid * _BPT, _BPT), _BPT), :], idx_v)

    def gather_descs(t, par):
        return [pltpu.make_async_copy(
                    aos_hbm.at[idx_v.at[t]],
                    rows_v.at[pl.ds(par * 128, 128), :], sem_g[par])]

    def out_descs(t, slot):
        row0 = pl.multiple_of((bt0 + t - 1) * 4, 8)
        return [pltpu.make_async_copy(
                    out_v.at[pl.ds((slot * 3 + i) * 8, 8), :],
                    out_hbm.at[pl.ds(i * (_NBLK * 4) + row0, 8), :],
                    sem_o[slot])
                for i in range(3)]

    def fire(descs):
        for d in descs:
            d.start()

    def drain(descs):
        for d in descs:
            d.wait()

    def compute(par, slot, half):
        for g in range(8):
            rows = par * 128 + g * _L + iota
            comp = [plsc.load_gather(rows_v,
                                     [rows, jnp.full((_L,), c, jnp.int32)])
                    for c in range(6)]
            tx, ty, tz, ax, ay, az = comp
            u = ax * ax + ay * ay + az * az
            cos = _horner(_COS_C, u)
            sinc = _horner(_SINC_C, u)
            omc = _horner(_OMC_C, u)
            s0 = sinc * ax
            s1 = sinc * ay
            s2 = sinc * az
            ox = omc * ax
            oy = omc * ay
            oz = omc * az
            vals = [
                [ox * ax + cos, ox * ay - s2, ox * az + s1, tx],
                [oy * ax + s2, oy * ay + cos, oy * az - s0, ty],
                [oz * ax - s1, oz * ay + s0, oz * az + cos, tz],
            ]
            for i in range(3):
                for j in range(4):
                    out_v[(slot * 3 + i) * 8 + half * 4 + j,
                          pl.ds(g * _L, _L)] = vals[i][j]

    fire(gather_descs(0, 0))

    def step(s, carry):
        for k in range(4):
            t = s * 4 + k
            par = k & 1
            slot = k >> 1
            half = k & 1
            if k == 3:
                @pl.when(s < (_BPT // 4) - 1)
                def _():
                    fire(gather_descs(t + 1, 0))
            else:
                fire(gather_descs(t + 1, (k + 1) & 1))
            drain(gather_descs(t, par))
            if k in (0, 2):
                @pl.when(s > 0)
                def _():
                    drain(out_descs((s - 1) * 4 + k + 1, slot))
            compute(par, slot, half)
            if k in (1, 3):
                fire(out_descs(t, slot))
        return carry

    lax.fori_loop(0, _BPT // 4, step, 0)
    last = (_BPT // 4 - 1) * 4
    drain(out_descs(last + 1, 0))
    drain(out_descs(last + 3, 1))


def kernel(frame_idx, bbox_idx, pose_adjustment):
    p = (lax.shift_left(lax.shift_right_logical(frame_idx, 3), 12)
         + lax.shift_left(lax.shift_right_logical(bbox_idx, 7), 10)
         + lax.shift_left(jnp.bitwise_and(frame_idx, 7), 7)
         + jnp.bitwise_and(bbox_idx, 127))
    soa = (jnp.transpose(pose_adjustment, (2, 0, 1))
           .reshape(6, 1250, 8, 4, 128)
           .transpose(0, 1, 3, 2, 4)
           .reshape(6 * _PLANE))
    mesh = plsc.VectorSubcoreMesh(
        core_axis_name="c", subcore_axis_name="s",
        num_cores=_NC, num_subcores=_NS)
    aos = pl.kernel(
        _convert_body,
        out_type=jax.ShapeDtypeStruct((_PLANE * 8,), jnp.float32),
        mesh=mesh,
        compiler_params=pltpu.CompilerParams(needs_layout_passes=False),
        scratch_types=[
            pltpu.VMEM((12, _CCH), jnp.float32),
            pltpu.VMEM((2, _CCH * 8), jnp.float32),
            pltpu.SemaphoreType.DMA,
            pltpu.SemaphoreType.DMA,
            pltpu.SemaphoreType.DMA,
            pltpu.SemaphoreType.DMA,
        ],
    )(soa)
    out = pl.kernel(
        _gather_body,
        out_type=jax.ShapeDtypeStruct((3 * _NBLK * 4, 128), jnp.float32),
        mesh=mesh,
        compiler_params=pltpu.CompilerParams(
            needs_layout_passes=False, use_tc_tiling_on_sc=False),
        scratch_types=[
            pltpu.VMEM((_BPT, 128), jnp.int32),
            pltpu.VMEM((256, 8), jnp.float32),
            pltpu.VMEM((48, 128), jnp.float32),
            pltpu.SemaphoreType.DMA,
            pltpu.SemaphoreType.DMA,
            pltpu.SemaphoreType.DMA,
            pltpu.SemaphoreType.DMA,
        ],
    )(p.reshape(_NBLK, 128), aos.reshape(_PLANE, 8))
    return (out.reshape(3, _NBLK, 4, 128)
               .transpose(1, 3, 0, 2)
               .reshape(_N, 3, 4))
